# baseline (speedup 1.0000x reference)
import functools

import jax
import jax.numpy as jnp
from jax import lax
from jax.experimental import pallas as pl
from jax.experimental.pallas import tpu as pltpu
from jax.experimental.pallas import tpu_sc as plsc

V = 65536
D = 256
B = 49152
NC, NS, L = 2, 16, 16
NW = NC * NS
BPW = B // NW
SEG = V // NS
SVR = SEG // L
WWIN = 18432
NWV = WWIN // L
GB = 128
NGB = BPW // GB
NBUF = 3
AHEAD = 1
IBUF = BPW + L

_mesh = plsc.VectorSubcoreMesh(core_axis_name="c", subcore_axis_name="s")
_params = pltpu.CompilerParams(needs_layout_passes=False)


@functools.partial(
    pl.kernel,
    out_type=jax.ShapeDtypeStruct((B, D), jnp.float32),
    mesh=_mesh,
    scratch_types=[
        pltpu.VMEM_SHARED((V + WWIN,), jnp.int32),
        pltpu.VMEM_SHARED((NS, L), jnp.int32),
        pltpu.VMEM((SEG,), jnp.int32),
        pltpu.VMEM((WWIN,), jnp.int32),
        pltpu.VMEM((NS, L), jnp.int32),
        pltpu.VMEM((L,), jnp.int32),
        pltpu.VMEM((IBUF,), jnp.int32),
        pltpu.VMEM((NBUF, GB, D), jnp.float32),
        [pltpu.SemaphoreType.DMA] * NBUF,
        [pltpu.SemaphoreType.DMA] * NBUF,
    ],
    compiler_params=_params,
)
def _compact_gather(mask_hbm, states_hbm, out_hbm, smask, scnt, segbuf, wbuf,
                    cnts, cntrow, ibuf, rowbuf, gsems, wsems):
    cid = lax.axis_index("c")
    sid = lax.axis_index("s")
    wid = sid * NC + cid
    lane = lax.iota(jnp.int32, L)
    target = jnp.full((L,), wid * BPW, jnp.int32)

    with jax.named_scope("stage"):
        pltpu.sync_copy(mask_hbm.at[pl.ds(sid * SEG, SEG)], segbuf)

        def cbody(c, acc):
            for k in range(8):
                acc = acc + plsc.all_reduce_population_count(
                    segbuf[pl.ds(c * (8 * L) + k * L, L)] > 0)
            return acc

        segcnt = lax.fori_loop(0, SVR // 8, cbody, jnp.zeros((L,), jnp.int32))
        pltpu.sync_copy(segbuf, smask.at[pl.ds(sid * SEG, SEG)])
        cntrow[pl.ds(0, L)] = segcnt
        pltpu.sync_copy(cntrow, scnt.at[sid])
        plsc.subcore_barrier()

    with jax.named_scope("locate"):
        pltpu.sync_copy(scnt, cnts)
        zero = jnp.zeros((L,), jnp.int32)
        acc, startseg, accseg = zero, zero, zero
        for s in range(NS):
            cnt = cnts[s, pl.ds(0, L)]
            take = (acc + cnt) <= target
            startseg = startseg + jnp.where(take, 1, 0)
            accseg = jnp.where(take, acc + cnt, accseg)
            acc = acc + cnt
        sstar = jnp.max(startseg)
        pltpu.sync_copy(smask.at[pl.ds(sstar * SEG, SEG)], segbuf)

        def lbody(c, carry):
            acc2, startv, accsel = carry
            for k in range(8):
                cnt = plsc.all_reduce_population_count(
                    segbuf[pl.ds(c * (8 * L) + k * L, L)] > 0)
                take = (acc2 + cnt) <= (target - accseg)
                startv = startv + jnp.where(take, 1, 0)
                accsel = jnp.where(take, acc2 + cnt, accsel)
                acc2 = acc2 + cnt
            return acc2, startv, accsel

        _, startv, accsel = lax.fori_loop(0, SVR // 8, lbody,
                                          (zero, zero, zero))
        k0 = target - accseg - accsel
        v0 = sstar * SVR + jnp.max(startv)
        a = v0 * L

    with jax.named_scope("walk"):
        pltpu.sync_copy(smask.at[pl.ds(a, WWIN)], wbuf)
        m0 = wbuf[pl.ds(0, L)]
        ison0 = m0 > 0
        one0 = jnp.where(ison0, 1, 0)
        pref0 = plsc.cumsum(one0) - one0
        keep0 = ison0 & (pref0 >= k0)
        plsc.store_compressed(ibuf.at[pl.ds(0, L)],
                              jnp.full((L,), a, jnp.int32) + lane, mask=keep0)
        coll0 = jnp.sum(jnp.where(keep0, 1, 0))

        def wcond(carry):
            coll, vi = carry
            return (coll < BPW) & (vi < NWV)

        def wbody(carry):
            coll, vi = carry
            m = wbuf[pl.ds(vi * L, L)]
            ison = m > 0
            pos = jnp.full((L,), a + vi * L, jnp.int32) + lane
            plsc.store_compressed(ibuf.at[pl.ds(coll, L)], pos, mask=ison)
            return coll + jnp.sum(jnp.where(ison, 1, 0)), vi + 1

        lax.while_loop(wcond, wbody, (coll0, jnp.int32(1)))

        vmax = jnp.full((L,), V - 1, jnp.int32)
        vmin = jnp.zeros((L,), jnp.int32)
        for t in range(IBUF // L):
            ibuf[pl.ds(t * L, L)] = jnp.clip(ibuf[pl.ds(t * L, L)], vmin,
                                             vmax)

    with jax.named_scope("gather"):
        obase = wid * BPW
        gh = [None] * NGB
        wh = [None] * NGB
        for j in range(AHEAD):
            gh[j] = pltpu.async_copy(
                states_hbm.at[ibuf.at[pl.ds(j * GB, GB)]],
                rowbuf.at[j % NBUF], gsems[j % NBUF])
        for j in range(NGB):
            b = j % NBUF
            gh[j].wait()
            wh[j] = pltpu.async_copy(
                rowbuf.at[b], out_hbm.at[pl.ds(obase + j * GB, GB)],
                wsems[b])
            nj = j + AHEAD
            if nj < NGB:
                nb = nj % NBUF
                if nj - NBUF >= 0:
                    wh[nj - NBUF].wait()
                gh[nj] = pltpu.async_copy(
                    states_hbm.at[ibuf.at[pl.ds(nj * GB, GB)]],
                    rowbuf.at[nb], gsems[nb])
        for j in range(NGB - NBUF, NGB):
            wh[j].wait()


def kernel(inputs, states, importance_scores, active_mask):
    return _compact_gather(active_mask.astype(jnp.int32), states)
